# baseline (speedup 1.0000x reference)
import functools

import jax
import jax.numpy as jnp
from jax import lax
from jax.experimental import pallas as pl
from jax.experimental.pallas import tpu as pltpu
from jax.experimental.pallas import tpu_sc as plsc

N_NODES = 10000
N_EDGES = 320000
D_FEAT = 128
N_HID1 = 128
N_HID2 = 64
N_TYPES = 86
N_TYPES_PAD = 96
N_PAIRS = 100000

NC = 2
NS = 16
NW = NC * NS
ECHUNK = 125
PCHUNK = 120
P_CHUNKS_W = 27
N_PAIRS_PAD = NW * P_CHUNKS_W * PCHUNK

E_ROWS_T = N_EDGES // ECHUNK // NS
E_ROWS_W = N_EDGES // ECHUNK // NW
P_ROWS_W = P_CHUNKS_W
NT_A = 624
NT_TAIL = N_NODES - NT_A * (NS - 1)

_mesh = plsc.VectorSubcoreMesh(core_axis_name="c", subcore_axis_name="s")
_sc_params = pltpu.CompilerParams(use_tc_tiling_on_sc=False)


def _worker_id():
  c = lax.axis_index("c")
  s = lax.axis_index("s")
  return c, s, c * NS + s


def _copy_node_rows(src, dst, s):
  @pl.when(s < NS - 1)
  def _():
    o = pl.multiple_of(s * NT_A, 8)
    pltpu.sync_copy(src.at[pl.ds(o, NT_A)], dst.at[pl.ds(o, NT_A)])
  @pl.when(s == NS - 1)
  def _():
    o = NT_A * (NS - 1)
    pltpu.sync_copy(src.at[pl.ds(o, NT_TAIL)], dst.at[pl.ds(o, NT_TAIL)])


@functools.partial(
    pl.kernel,
    out_type=jax.ShapeDtypeStruct((NC, N_NODES, 16), jnp.float32),
    mesh=_mesh,
    scratch_types=[
        pltpu.VMEM((E_ROWS_W, ECHUNK), jnp.int32),
        pltpu.VMEM((ECHUNK, 16), jnp.float32),
        pltpu.VMEM_SHARED((N_NODES, 16), jnp.float32),
    ],
    compiler_params=_sc_params,
)
def _sc_degree(dst_hbm, ones_hbm, zeros_hbm, out_hbm, idx_v, ones_v, acc):
  c, s, wid = _worker_id()
  _copy_node_rows(zeros_hbm, acc, s)
  pltpu.sync_copy(ones_hbm, ones_v)
  pltpu.sync_copy(dst_hbm.at[wid], idx_v)
  plsc.subcore_barrier()

  def body(j, _):
    pltpu.sync_copy(ones_v, acc.at[idx_v.at[j]], add=True)
    return ()
  lax.fori_loop(0, E_ROWS_W, body, ())

  plsc.subcore_barrier()
  _copy_node_rows(acc, out_hbm.at[c], s)


def _make_sc_segsum(d):
  @functools.partial(
      pl.kernel,
      out_type=(jax.ShapeDtypeStruct((N_NODES, d), jnp.float32),
                jax.ShapeDtypeStruct((N_NODES, d), jnp.float32)),
      mesh=_mesh,
      scratch_types=[
          pltpu.VMEM((E_ROWS_T, ECHUNK), jnp.int32),
          pltpu.VMEM((E_ROWS_T, ECHUNK), jnp.int32),
          pltpu.VMEM((ECHUNK, d), jnp.float32),
          pltpu.VMEM((ECHUNK, d), jnp.float32),
          pltpu.VMEM((ECHUNK, d), jnp.float32),
          pltpu.VMEM((ECHUNK, d), jnp.float32),
          pltpu.VMEM_SHARED((N_NODES, d), jnp.float32),
          pltpu.SemaphoreType.DMA,
          pltpu.SemaphoreType.DMA,
          pltpu.SemaphoreType.DMA,
          pltpu.SemaphoreType.DMA,
          pltpu.SemaphoreType.DMA,
          pltpu.SemaphoreType.DMA,
          pltpu.SemaphoreType.DMA,
          pltpu.SemaphoreType.DMA,
      ],
      compiler_params=_sc_params,
  )
  def segsum(ga_hbm, gb_hbm, src_hbm, dst_hbm, zeros_hbm, outa_hbm, outb_hbm,
             src_v, dst_v, buf0, buf1, buf2, buf3, acc,
             gs0, gs1, gs2, gs3, ss0, ss1, ss2, ss3):
    c, s, wid = _worker_id()
    bufs = (buf0, buf1, buf2, buf3)
    gsems = (gs0, gs1, gs2, gs3)
    ssems = (ss0, ss1, ss2, ss3)
    _copy_node_rows(zeros_hbm, acc, s)
    pltpu.sync_copy(src_hbm.at[s], src_v)
    pltpu.sync_copy(dst_hbm.at[s], dst_v)
    plsc.subcore_barrier()

    def run(g_hbm):
      for b in range(3):
        pltpu.async_copy(g_hbm.at[src_v.at[b]], bufs[b], gsems[b])

      def body(k, _):
        for b in range(4):
          j = 4 * k + b
          b3 = (b + 3) % 4
          pltpu.make_async_copy(g_hbm.at[src_v.at[j]], bufs[b],
                                gsems[b]).wait()
          pltpu.async_copy(bufs[b], acc.at[dst_v.at[j]], ssems[b], add=True)
          @pl.when(j + 3 < E_ROWS_T)
          def _():
            @pl.when(j >= 1)
            def _():
              pltpu.make_async_copy(bufs[b3], acc.at[dst_v.at[j - 1]],
                                    ssems[b3]).wait()
            pltpu.async_copy(g_hbm.at[src_v.at[j + 3]], bufs[b3], gsems[b3])
        return ()
      lax.fori_loop(0, E_ROWS_T // 4, body, ())
      for b in range(4):
        j = E_ROWS_T - 4 + b
        pltpu.make_async_copy(bufs[b], acc.at[dst_v.at[j]], ssems[b]).wait()

    @pl.when(c == 0)
    def _():
      run(ga_hbm)
    @pl.when(c == 1)
    def _():
      run(gb_hbm)

    plsc.subcore_barrier()
    @pl.when(c == 0)
    def _():
      _copy_node_rows(acc, outa_hbm, s)
    @pl.when(c == 1)
    def _():
      _copy_node_rows(acc, outb_hbm, s)
  return segsum


_sc_segsum_64 = _make_sc_segsum(D_FEAT // 2)
_sc_segsum_32 = _make_sc_segsum(N_HID2 // 2)


@functools.partial(
    pl.kernel,
    out_type=jax.ShapeDtypeStruct((NW, P_ROWS_W, PCHUNK, N_TYPES_PAD),
                                  jnp.float32),
    mesh=_mesh,
    scratch_types=[
        pltpu.VMEM((P_ROWS_W, PCHUNK), jnp.int32),
        pltpu.VMEM((P_ROWS_W, PCHUNK), jnp.int32),
        pltpu.VMEM((PCHUNK, N_TYPES_PAD), jnp.float32),
        pltpu.VMEM((PCHUNK, N_TYPES_PAD), jnp.float32),
        pltpu.VMEM((PCHUNK, N_TYPES_PAD), jnp.float32),
        pltpu.VMEM((PCHUNK, N_TYPES_PAD), jnp.float32),
        pltpu.VMEM((PCHUNK, N_TYPES_PAD), jnp.float32),
        pltpu.VMEM((PCHUNK, N_TYPES_PAD), jnp.float32),
        pltpu.SemaphoreType.DMA,
        pltpu.SemaphoreType.DMA,
        pltpu.SemaphoreType.DMA,
        pltpu.SemaphoreType.DMA,
        pltpu.SemaphoreType.DMA,
        pltpu.SemaphoreType.DMA,
    ],
    compiler_params=_sc_params,
)
def _sc_pairs(p1_hbm, p2_hbm, d1_hbm, d2_hbm, out_hbm,
              i1_v, i2_v, r1a, r2a, r1b, r2b, oba, obb,
              g1a, g2a, g1b, g2b, wsa, wsb):
  c, s, wid = _worker_id()
  pltpu.sync_copy(d1_hbm.at[wid], i1_v)
  pltpu.sync_copy(d2_hbm.at[wid], i2_v)

  def gather(j, r1, r2, g1, g2):
    pltpu.async_copy(p1_hbm.at[i1_v.at[j]], r1, g1)
    pltpu.async_copy(p2_hbm.at[i2_v.at[j]], r2, g2)

  def slot(j, r1, r2, g1, g2, ob, ws, r1n, r2n, g1n, g2n):
    pltpu.make_async_copy(p1_hbm.at[i1_v.at[j]], r1, g1).wait()
    pltpu.make_async_copy(p2_hbm.at[i2_v.at[j]], r2, g2).wait()
    @pl.when(j + 1 < P_ROWS_W)
    def _():
      gather(j + 1, r1n, r2n, g1n, g2n)
    @pl.when(j >= 2)
    def _():
      pltpu.make_async_copy(ob, out_hbm.at[wid, j - 2], ws).wait()

    def crow(r, _):
      for cc in range(N_TYPES_PAD // 16):
        a = r1[r, pl.ds(cc * 16, 16)]
        b = r2[r, pl.ds(cc * 16, 16)]
        z = a + b
        ob[r, pl.ds(cc * 16, 16)] = 1.0 / (1.0 + jnp.exp(-z))
      return ()
    lax.fori_loop(0, PCHUNK, crow, ())
    pltpu.async_copy(ob, out_hbm.at[wid, j], ws)

  gather(0, r1a, r2a, g1a, g2a)

  def body(k, _):
    j0 = 2 * k
    slot(j0, r1a, r2a, g1a, g2a, oba, wsa, r1b, r2b, g1b, g2b)
    slot(j0 + 1, r1b, r2b, g1b, g2b, obb, wsb, r1a, r2a, g1a, g2a)
    return ()
  lax.fori_loop(0, P_ROWS_W // 2, body, ())
  slot(P_ROWS_W - 1, r1a, r2a, g1a, g2a, oba, wsa, r1b, r2b, g1b, g2b)
  pltpu.make_async_copy(obb, out_hbm.at[wid, P_ROWS_W - 2], wsb).wait()
  pltpu.make_async_copy(oba, out_hbm.at[wid, P_ROWS_W - 1], wsa).wait()


_BLK = 1000
_GRID = N_NODES // _BLK
_H = D_FEAT // 2
_Q = N_HID2 // 2


def _tc1_body(x_ref, w1_ref, p0_ref, p1_ref, ga_ref, gb_ref, dinv_ref):
  deg = 1.0 + p0_ref[:, 0:1] + p1_ref[:, 0:1]
  dinv = lax.rsqrt(deg)
  h = jnp.dot(x_ref[...], w1_ref[...], preferred_element_type=jnp.float32)
  g = h * dinv
  ga_ref[...] = g[:, :_H]
  gb_ref[...] = g[:, _H:]
  dinv_ref[...] = jnp.broadcast_to(dinv, (_BLK, D_FEAT))


def _tc2_body(sa_ref, sb_ref, ga_ref, gb_ref, dinv_ref, b1_ref, w2_ref,
              g2a_ref, g2b_ref):
  s = jnp.concatenate([sa_ref[...], sb_ref[...]], axis=1)
  g = jnp.concatenate([ga_ref[...], gb_ref[...]], axis=1)
  pre = dinv_ref[...] * (s + g) + b1_ref[...]
  h1 = jnp.maximum(pre, 0.0)
  g2 = jnp.dot(h1, w2_ref[...], preferred_element_type=jnp.float32)
  g2 = g2 * dinv_ref[:, :N_HID2]
  g2a_ref[...] = g2[:, :_Q]
  g2b_ref[...] = g2[:, _Q:]


def _tc3_body(sa_ref, sb_ref, g2a_ref, g2b_ref, dinv_ref, b2_ref,
              wf1_ref, wf2_ref, bf_ref, o1_ref, o2_ref):
  s = jnp.concatenate([sa_ref[...], sb_ref[...]], axis=1)
  g = jnp.concatenate([g2a_ref[...], g2b_ref[...]], axis=1)
  pre = dinv_ref[:, :N_HID2] * (s + g) + b2_ref[...]
  h2 = jnp.maximum(pre, 0.0)
  o1_ref[...] = jnp.dot(h2, wf1_ref[...],
                        preferred_element_type=jnp.float32) + bf_ref[...]
  o2_ref[...] = jnp.dot(h2, wf2_ref[...], preferred_element_type=jnp.float32)


def _row_spec(d):
  return pl.BlockSpec((_BLK, d), lambda i: (i, 0))


def _full_spec(r, d):
  return pl.BlockSpec((r, d), lambda i: (0, 0))


_tc1 = pl.pallas_call(
    _tc1_body,
    grid=(_GRID,),
    in_specs=[_row_spec(D_FEAT), _full_spec(D_FEAT, N_HID1),
              _row_spec(16), _row_spec(16)],
    out_specs=[_row_spec(_H), _row_spec(_H), _row_spec(D_FEAT)],
    out_shape=[jax.ShapeDtypeStruct((N_NODES, _H), jnp.float32),
               jax.ShapeDtypeStruct((N_NODES, _H), jnp.float32),
               jax.ShapeDtypeStruct((N_NODES, D_FEAT), jnp.float32)],
)

_tc2 = pl.pallas_call(
    _tc2_body,
    grid=(_GRID,),
    in_specs=[_row_spec(_H), _row_spec(_H), _row_spec(_H), _row_spec(_H),
              _row_spec(D_FEAT), _full_spec(1, N_HID1),
              _full_spec(N_HID1, N_HID2)],
    out_specs=[_row_spec(_Q), _row_spec(_Q)],
    out_shape=[jax.ShapeDtypeStruct((N_NODES, _Q), jnp.float32),
               jax.ShapeDtypeStruct((N_NODES, _Q), jnp.float32)],
)

_tc3 = pl.pallas_call(
    _tc3_body,
    grid=(_GRID,),
    in_specs=[_row_spec(_Q), _row_spec(_Q), _row_spec(_Q), _row_spec(_Q),
              _row_spec(D_FEAT),
              _full_spec(1, N_HID2),
              _full_spec(N_HID2, N_TYPES_PAD), _full_spec(N_HID2, N_TYPES_PAD),
              _full_spec(1, N_TYPES_PAD)],
    out_specs=[_row_spec(N_TYPES_PAD), _row_spec(N_TYPES_PAD)],
    out_shape=[jax.ShapeDtypeStruct((N_NODES, N_TYPES_PAD), jnp.float32),
               jax.ShapeDtypeStruct((N_NODES, N_TYPES_PAD), jnp.float32)],
)


@jax.jit
def kernel(x, W1, b1, W2, b2, Wf, bf, edge_index, drug1_idx, drug2_idx):
  dst_w = edge_index[1].reshape(NW, E_ROWS_W, ECHUNK)
  src_t = edge_index[0].reshape(NS, E_ROWS_T, ECHUNK)
  dst_t = edge_index[1].reshape(NS, E_ROWS_T, ECHUNK)
  npad = N_PAIRS_PAD - N_PAIRS
  d1_3d = jnp.pad(drug1_idx, (0, npad)).reshape(NW, P_ROWS_W, PCHUNK)
  d2_3d = jnp.pad(drug2_idx, (0, npad)).reshape(NW, P_ROWS_W, PCHUNK)

  ones16 = jnp.ones((ECHUNK, 16), jnp.float32)
  zeros16 = jnp.zeros((N_NODES, 16), jnp.float32)
  zeros_h = jnp.zeros((N_NODES, _H), jnp.float32)
  zeros_q = jnp.zeros((N_NODES, _Q), jnp.float32)

  wf1 = jnp.zeros((N_HID2, N_TYPES_PAD), jnp.float32).at[:, :N_TYPES].set(Wf[:N_HID2])
  wf2 = jnp.zeros((N_HID2, N_TYPES_PAD), jnp.float32).at[:, :N_TYPES].set(Wf[N_HID2:])
  bfp = jnp.zeros((1, N_TYPES_PAD), jnp.float32).at[0, :N_TYPES].set(bf)

  degp = _sc_degree(dst_w, ones16, zeros16)
  ga, gb, dinvb = _tc1(x, W1, degp[0], degp[1])
  s1a, s1b = _sc_segsum_64(ga, gb, src_t, dst_t, zeros_h)
  g2a, g2b = _tc2(s1a, s1b, ga, gb, dinvb, b1.reshape(1, N_HID1), W2)
  s2a, s2b = _sc_segsum_32(g2a, g2b, src_t, dst_t, zeros_q)
  p1, p2 = _tc3(s2a, s2b, g2a, g2b, dinvb, b2.reshape(1, N_HID2), wf1, wf2, bfp)
  out = _sc_pairs(p1, p2, d1_3d, d2_3d)
  return out.reshape(N_PAIRS_PAD, N_TYPES_PAD)[:N_PAIRS, :N_TYPES]


# speedup vs baseline: 1.3236x; 1.3236x over previous
import functools

import jax
import jax.numpy as jnp
from jax import lax
from jax.experimental import pallas as pl
from jax.experimental.pallas import tpu as pltpu
from jax.experimental.pallas import tpu_sc as plsc

N_NODES = 10000
N_EDGES = 320000
D_FEAT = 128
N_HID1 = 128
N_HID2 = 64
N_TYPES = 86
N_TYPES_PAD = 96
N_PAIRS = 100000

NC = 2
NS = 16
NW = NC * NS
ECHUNK = 125
PCHUNK = 120
P_CHUNKS_W = 27
N_PAIRS_PAD = NW * P_CHUNKS_W * PCHUNK

E_ROWS_T = N_EDGES // ECHUNK // NS
E_ROWS_W = N_EDGES // ECHUNK // NW
P_ROWS_W = P_CHUNKS_W
NT_A = 624
NT_TAIL = N_NODES - NT_A * (NS - 1)

_mesh = plsc.VectorSubcoreMesh(core_axis_name="c", subcore_axis_name="s")
_sc_params = pltpu.CompilerParams(use_tc_tiling_on_sc=False)


def _worker_id():
  c = lax.axis_index("c")
  s = lax.axis_index("s")
  return c, s, c * NS + s


def _copy_node_rows(src, dst, s):
  @pl.when(s < NS - 1)
  def _():
    o = pl.multiple_of(s * NT_A, 8)
    pltpu.sync_copy(src.at[pl.ds(o, NT_A)], dst.at[pl.ds(o, NT_A)])
  @pl.when(s == NS - 1)
  def _():
    o = NT_A * (NS - 1)
    pltpu.sync_copy(src.at[pl.ds(o, NT_TAIL)], dst.at[pl.ds(o, NT_TAIL)])


@functools.partial(
    pl.kernel,
    out_type=jax.ShapeDtypeStruct((NC, N_NODES, 16), jnp.float32),
    mesh=_mesh,
    scratch_types=[
        pltpu.VMEM((E_ROWS_W, ECHUNK), jnp.int32),
        pltpu.VMEM((ECHUNK, 16), jnp.float32),
        pltpu.VMEM_SHARED((N_NODES, 16), jnp.float32),
    ],
    compiler_params=_sc_params,
)
def _sc_degree(dst_hbm, ones_hbm, zeros_hbm, out_hbm, idx_v, ones_v, acc):
  c, s, wid = _worker_id()
  _copy_node_rows(zeros_hbm, acc, s)
  pltpu.sync_copy(ones_hbm, ones_v)
  pltpu.sync_copy(dst_hbm.at[wid], idx_v)
  plsc.subcore_barrier()

  def body(j, _):
    pltpu.sync_copy(ones_v, acc.at[idx_v.at[j]], add=True)
    return ()
  lax.fori_loop(0, E_ROWS_W, body, ())

  plsc.subcore_barrier()
  _copy_node_rows(acc, out_hbm.at[c], s)


def _make_sc_segsum(d):
  @functools.partial(
      pl.kernel,
      out_type=(jax.ShapeDtypeStruct((N_NODES, d), jnp.float32),
                jax.ShapeDtypeStruct((N_NODES, d), jnp.float32)),
      mesh=_mesh,
      scratch_types=[
          pltpu.VMEM((E_ROWS_T, ECHUNK), jnp.int32),
          pltpu.VMEM((E_ROWS_T, ECHUNK), jnp.int32),
          pltpu.VMEM((ECHUNK, d), jnp.float32),
          pltpu.VMEM((ECHUNK, d), jnp.float32),
          pltpu.VMEM((ECHUNK, d), jnp.float32),
          pltpu.VMEM((ECHUNK, d), jnp.float32),
          pltpu.VMEM_SHARED((N_NODES, d), jnp.float32),
          pltpu.SemaphoreType.DMA,
          pltpu.SemaphoreType.DMA,
          pltpu.SemaphoreType.DMA,
          pltpu.SemaphoreType.DMA,
          pltpu.SemaphoreType.DMA,
          pltpu.SemaphoreType.DMA,
          pltpu.SemaphoreType.DMA,
          pltpu.SemaphoreType.DMA,
      ],
      compiler_params=_sc_params,
  )
  def segsum(ga_hbm, gb_hbm, src_hbm, dst_hbm, zeros_hbm, outa_hbm, outb_hbm,
             src_v, dst_v, buf0, buf1, buf2, buf3, acc,
             gs0, gs1, gs2, gs3, ss0, ss1, ss2, ss3):
    c, s, wid = _worker_id()
    bufs = (buf0, buf1, buf2, buf3)
    gsems = (gs0, gs1, gs2, gs3)
    ssems = (ss0, ss1, ss2, ss3)
    _copy_node_rows(zeros_hbm, acc, s)
    pltpu.sync_copy(src_hbm.at[s], src_v)
    pltpu.sync_copy(dst_hbm.at[s], dst_v)
    plsc.subcore_barrier()

    def run(g_hbm):
      for b in range(3):
        pltpu.async_copy(g_hbm.at[src_v.at[b]], bufs[b], gsems[b])

      def body(k, _):
        for b in range(4):
          j = 4 * k + b
          b3 = (b + 3) % 4
          pltpu.make_async_copy(g_hbm.at[src_v.at[j]], bufs[b],
                                gsems[b]).wait()
          pltpu.async_copy(bufs[b], acc.at[dst_v.at[j]], ssems[b], add=True)
          @pl.when(j + 3 < E_ROWS_T)
          def _():
            @pl.when(j >= 1)
            def _():
              pltpu.make_async_copy(bufs[b3], acc.at[dst_v.at[j - 1]],
                                    ssems[b3]).wait()
            pltpu.async_copy(g_hbm.at[src_v.at[j + 3]], bufs[b3], gsems[b3])
        return ()
      lax.fori_loop(0, E_ROWS_T // 4, body, ())
      for b in range(4):
        j = E_ROWS_T - 4 + b
        pltpu.make_async_copy(bufs[b], acc.at[dst_v.at[j]], ssems[b]).wait()

    @pl.when(c == 0)
    def _():
      run(ga_hbm)
    @pl.when(c == 1)
    def _():
      run(gb_hbm)

    plsc.subcore_barrier()
    @pl.when(c == 0)
    def _():
      _copy_node_rows(acc, outa_hbm, s)
    @pl.when(c == 1)
    def _():
      _copy_node_rows(acc, outb_hbm, s)
  return segsum


_sc_segsum_64 = _make_sc_segsum(D_FEAT // 2)
_sc_segsum_32 = _make_sc_segsum(N_HID2 // 2)


@functools.partial(
    pl.kernel,
    out_type=jax.ShapeDtypeStruct((NW, P_ROWS_W, PCHUNK, N_TYPES_PAD),
                                  jnp.float32),
    mesh=_mesh,
    scratch_types=[
        pltpu.VMEM((P_ROWS_W, PCHUNK), jnp.int32),
        pltpu.VMEM((P_ROWS_W, PCHUNK), jnp.int32),
        pltpu.VMEM((PCHUNK, N_TYPES_PAD), jnp.float32),
        pltpu.VMEM((PCHUNK, N_TYPES_PAD), jnp.float32),
        pltpu.VMEM((PCHUNK, N_TYPES_PAD), jnp.float32),
        pltpu.VMEM((PCHUNK, N_TYPES_PAD), jnp.float32),
        pltpu.VMEM((PCHUNK, N_TYPES_PAD), jnp.float32),
        pltpu.VMEM((PCHUNK, N_TYPES_PAD), jnp.float32),
        pltpu.SemaphoreType.DMA,
        pltpu.SemaphoreType.DMA,
        pltpu.SemaphoreType.DMA,
        pltpu.SemaphoreType.DMA,
        pltpu.SemaphoreType.DMA,
        pltpu.SemaphoreType.DMA,
    ],
    compiler_params=_sc_params,
)
def _sc_pairs(p1_hbm, p2_hbm, d1_hbm, d2_hbm, out_hbm,
              i1_v, i2_v, r1a, r2a, r1b, r2b, oba, obb,
              g1a, g2a, g1b, g2b, wsa, wsb):
  c, s, wid = _worker_id()
  pltpu.sync_copy(d1_hbm.at[wid], i1_v)
  pltpu.sync_copy(d2_hbm.at[wid], i2_v)

  def gather(j, r1, r2, g1, g2):
    pltpu.async_copy(p1_hbm.at[i1_v.at[j]], r1, g1)
    pltpu.async_copy(p2_hbm.at[i2_v.at[j]], r2, g2)

  def slot(j, r1, r2, g1, g2, ob, ws, r1n, r2n, g1n, g2n):
    pltpu.make_async_copy(p1_hbm.at[i1_v.at[j]], r1, g1).wait()
    pltpu.make_async_copy(p2_hbm.at[i2_v.at[j]], r2, g2).wait()
    @pl.when(j + 1 < P_ROWS_W)
    def _():
      gather(j + 1, r1n, r2n, g1n, g2n)
    @pl.when(j >= 2)
    def _():
      pltpu.make_async_copy(ob, out_hbm.at[wid, j - 2], ws).wait()

    def crow(r, _):
      for cc in range(N_TYPES_PAD // 16):
        a = r1[r, pl.ds(cc * 16, 16)]
        b = r2[r, pl.ds(cc * 16, 16)]
        z = a + b
        ob[r, pl.ds(cc * 16, 16)] = 1.0 / (1.0 + jnp.exp(-z))
      return ()
    lax.fori_loop(0, PCHUNK, crow, ())
    pltpu.async_copy(ob, out_hbm.at[wid, j], ws)

  gather(0, r1a, r2a, g1a, g2a)

  def body(k, _):
    j0 = 2 * k
    slot(j0, r1a, r2a, g1a, g2a, oba, wsa, r1b, r2b, g1b, g2b)
    slot(j0 + 1, r1b, r2b, g1b, g2b, obb, wsb, r1a, r2a, g1a, g2a)
    return ()
  lax.fori_loop(0, P_ROWS_W // 2, body, ())
  slot(P_ROWS_W - 1, r1a, r2a, g1a, g2a, oba, wsa, r1b, r2b, g1b, g2b)
  pltpu.make_async_copy(obb, out_hbm.at[wid, P_ROWS_W - 2], wsb).wait()
  pltpu.make_async_copy(oba, out_hbm.at[wid, P_ROWS_W - 1], wsa).wait()


_BLK = 1000
_GRID = N_NODES // _BLK
_H = D_FEAT // 2
_Q = N_HID2 // 2


def _tc1_body(x_ref, w1_ref, p0_ref, p1_ref, ga_ref, gb_ref, dinv_ref):
  deg = 1.0 + p0_ref[:, 0:1] + p1_ref[:, 0:1]
  dinv = lax.rsqrt(deg)
  h = jnp.dot(x_ref[...], w1_ref[...], preferred_element_type=jnp.float32)
  g = h * dinv
  ga_ref[...] = g[:, :_H]
  gb_ref[...] = g[:, _H:]
  dinv_ref[...] = jnp.broadcast_to(dinv, (_BLK, D_FEAT))


def _tc2_body(sa_ref, sb_ref, ga_ref, gb_ref, dinv_ref, b1_ref, w2_ref,
              g2a_ref, g2b_ref):
  s = jnp.concatenate([sa_ref[...], sb_ref[...]], axis=1)
  g = jnp.concatenate([ga_ref[...], gb_ref[...]], axis=1)
  pre = dinv_ref[...] * (s + g) + b1_ref[...]
  h1 = jnp.maximum(pre, 0.0)
  g2 = jnp.dot(h1, w2_ref[...], preferred_element_type=jnp.float32)
  g2 = g2 * dinv_ref[:, :N_HID2]
  g2a_ref[...] = g2[:, :_Q]
  g2b_ref[...] = g2[:, _Q:]


def _tc3_body(sa_ref, sb_ref, g2a_ref, g2b_ref, dinv_ref, b2_ref,
              wf1_ref, wf2_ref, bf_ref, o1_ref, o2_ref):
  s = jnp.concatenate([sa_ref[...], sb_ref[...]], axis=1)
  g = jnp.concatenate([g2a_ref[...], g2b_ref[...]], axis=1)
  pre = dinv_ref[:, :N_HID2] * (s + g) + b2_ref[...]
  h2 = jnp.maximum(pre, 0.0)
  o1_ref[...] = jnp.dot(h2, wf1_ref[...],
                        preferred_element_type=jnp.float32) + bf_ref[...]
  o2_ref[...] = jnp.dot(h2, wf2_ref[...], preferred_element_type=jnp.float32)


def _row_spec(d):
  return pl.BlockSpec((_BLK, d), lambda i: (i, 0))


def _full_spec(r, d):
  return pl.BlockSpec((r, d), lambda i: (0, 0))


_tc1 = pl.pallas_call(
    _tc1_body,
    grid=(_GRID,),
    in_specs=[_row_spec(D_FEAT), _full_spec(D_FEAT, N_HID1),
              _row_spec(16), _row_spec(16)],
    out_specs=[_row_spec(_H), _row_spec(_H), _row_spec(D_FEAT)],
    out_shape=[jax.ShapeDtypeStruct((N_NODES, _H), jnp.float32),
               jax.ShapeDtypeStruct((N_NODES, _H), jnp.float32),
               jax.ShapeDtypeStruct((N_NODES, D_FEAT), jnp.float32)],
)

_tc2 = pl.pallas_call(
    _tc2_body,
    grid=(_GRID,),
    in_specs=[_row_spec(_H), _row_spec(_H), _row_spec(_H), _row_spec(_H),
              _row_spec(D_FEAT), _full_spec(1, N_HID1),
              _full_spec(N_HID1, N_HID2)],
    out_specs=[_row_spec(_Q), _row_spec(_Q)],
    out_shape=[jax.ShapeDtypeStruct((N_NODES, _Q), jnp.float32),
               jax.ShapeDtypeStruct((N_NODES, _Q), jnp.float32)],
)

_tc3 = pl.pallas_call(
    _tc3_body,
    grid=(_GRID,),
    in_specs=[_row_spec(_Q), _row_spec(_Q), _row_spec(_Q), _row_spec(_Q),
              _row_spec(D_FEAT),
              _full_spec(1, N_HID2),
              _full_spec(N_HID2, N_TYPES_PAD), _full_spec(N_HID2, N_TYPES_PAD),
              _full_spec(1, N_TYPES_PAD)],
    out_specs=[_row_spec(N_TYPES_PAD), _row_spec(N_TYPES_PAD)],
    out_shape=[jax.ShapeDtypeStruct((N_NODES, N_TYPES_PAD), jnp.float32),
               jax.ShapeDtypeStruct((N_NODES, N_TYPES_PAD), jnp.float32)],
)


@jax.jit
def kernel(x, W1, b1, W2, b2, Wf, bf, edge_index, drug1_idx, drug2_idx):
  dst_w = edge_index[1].reshape(NW, E_ROWS_W, ECHUNK)
  src_t = edge_index[0].reshape(NS, E_ROWS_T, ECHUNK)
  dst_t = edge_index[1].reshape(NS, E_ROWS_T, ECHUNK)
  npad = N_PAIRS_PAD - N_PAIRS
  d1_3d = jnp.pad(drug1_idx, (0, npad), mode="wrap").reshape(NW, P_ROWS_W, PCHUNK)
  d2_3d = jnp.pad(drug2_idx, (0, npad), mode="wrap").reshape(NW, P_ROWS_W, PCHUNK)

  ones16 = jnp.ones((ECHUNK, 16), jnp.float32)
  zeros16 = jnp.zeros((N_NODES, 16), jnp.float32)
  zeros_h = jnp.zeros((N_NODES, _H), jnp.float32)
  zeros_q = jnp.zeros((N_NODES, _Q), jnp.float32)

  wf1 = jnp.zeros((N_HID2, N_TYPES_PAD), jnp.float32).at[:, :N_TYPES].set(Wf[:N_HID2])
  wf2 = jnp.zeros((N_HID2, N_TYPES_PAD), jnp.float32).at[:, :N_TYPES].set(Wf[N_HID2:])
  bfp = jnp.zeros((1, N_TYPES_PAD), jnp.float32).at[0, :N_TYPES].set(bf)

  degp = _sc_degree(dst_w, ones16, zeros16)
  ga, gb, dinvb = _tc1(x, W1, degp[0], degp[1])
  s1a, s1b = _sc_segsum_64(ga, gb, src_t, dst_t, zeros_h)
  g2a, g2b = _tc2(s1a, s1b, ga, gb, dinvb, b1.reshape(1, N_HID1), W2)
  s2a, s2b = _sc_segsum_32(g2a, g2b, src_t, dst_t, zeros_q)
  p1, p2 = _tc3(s2a, s2b, g2a, g2b, dinvb, b2.reshape(1, N_HID2), wf1, wf2, bfp)
  out = _sc_pairs(p1, p2, d1_3d, d2_3d)
  return out.reshape(N_PAIRS_PAD, N_TYPES_PAD)[:N_PAIRS, :N_TYPES]


# speedup vs baseline: 1.4293x; 1.0799x over previous
import functools

import jax
import jax.numpy as jnp
from jax import lax
from jax.experimental import pallas as pl
from jax.experimental.pallas import tpu as pltpu
from jax.experimental.pallas import tpu_sc as plsc

N_NODES = 10000
N_EDGES = 320000
D_FEAT = 128
N_HID1 = 128
N_HID2 = 64
N_TYPES = 86
N_TYPES_PAD = 96
N_PAIRS = 100000

NC = 2
NS = 16
NW = NC * NS
ECHUNK = 125
PCHUNK = 125
P_CHUNKS_W = 25

E_ROWS_T = N_EDGES // ECHUNK // NS
E_ROWS_W = N_EDGES // ECHUNK // NW
P_ROWS_W = P_CHUNKS_W
NT_A = 624
NT_TAIL = N_NODES - NT_A * (NS - 1)

_mesh = plsc.VectorSubcoreMesh(core_axis_name="c", subcore_axis_name="s")
_sc_params = pltpu.CompilerParams(use_tc_tiling_on_sc=False)


def _worker_id():
  c = lax.axis_index("c")
  s = lax.axis_index("s")
  return c, s, c * NS + s


def _copy_node_rows(src, dst, s):
  @pl.when(s < NS - 1)
  def _():
    o = pl.multiple_of(s * NT_A, 8)
    pltpu.sync_copy(src.at[pl.ds(o, NT_A)], dst.at[pl.ds(o, NT_A)])
  @pl.when(s == NS - 1)
  def _():
    o = NT_A * (NS - 1)
    pltpu.sync_copy(src.at[pl.ds(o, NT_TAIL)], dst.at[pl.ds(o, NT_TAIL)])


@functools.partial(
    pl.kernel,
    out_type=jax.ShapeDtypeStruct((NC, N_NODES, 16), jnp.float32),
    mesh=_mesh,
    scratch_types=[
        pltpu.VMEM((E_ROWS_W, ECHUNK), jnp.int32),
        pltpu.VMEM((ECHUNK, 16), jnp.float32),
        pltpu.VMEM_SHARED((N_NODES, 16), jnp.float32),
    ],
    compiler_params=_sc_params,
)
def _sc_degree(dst_hbm, ones_hbm, zeros_hbm, out_hbm, idx_v, ones_v, acc):
  c, s, wid = _worker_id()
  _copy_node_rows(zeros_hbm, acc, s)
  pltpu.sync_copy(ones_hbm, ones_v)
  pltpu.sync_copy(dst_hbm.at[wid], idx_v)
  plsc.subcore_barrier()

  def body(j, _):
    pltpu.sync_copy(ones_v, acc.at[idx_v.at[j]], add=True)
    return ()
  lax.fori_loop(0, E_ROWS_W, body, ())

  plsc.subcore_barrier()
  _copy_node_rows(acc, out_hbm.at[c], s)


def _make_sc_segsum(d):
  @functools.partial(
      pl.kernel,
      out_type=(jax.ShapeDtypeStruct((N_NODES, d), jnp.float32),
                jax.ShapeDtypeStruct((N_NODES, d), jnp.float32)),
      mesh=_mesh,
      scratch_types=[
          pltpu.VMEM((E_ROWS_T, ECHUNK), jnp.int32),
          pltpu.VMEM((E_ROWS_T, ECHUNK), jnp.int32),
          pltpu.VMEM((ECHUNK, d), jnp.float32),
          pltpu.VMEM((ECHUNK, d), jnp.float32),
          pltpu.VMEM((ECHUNK, d), jnp.float32),
          pltpu.VMEM((ECHUNK, d), jnp.float32),
          pltpu.VMEM_SHARED((N_NODES, d), jnp.float32),
          pltpu.SemaphoreType.DMA,
          pltpu.SemaphoreType.DMA,
          pltpu.SemaphoreType.DMA,
          pltpu.SemaphoreType.DMA,
          pltpu.SemaphoreType.DMA,
          pltpu.SemaphoreType.DMA,
          pltpu.SemaphoreType.DMA,
          pltpu.SemaphoreType.DMA,
      ],
      compiler_params=_sc_params,
  )
  def segsum(ga_hbm, gb_hbm, src_hbm, dst_hbm, zeros_hbm, outa_hbm, outb_hbm,
             src_v, dst_v, buf0, buf1, buf2, buf3, acc,
             gs0, gs1, gs2, gs3, ss0, ss1, ss2, ss3):
    c, s, wid = _worker_id()
    bufs = (buf0, buf1, buf2, buf3)
    gsems = (gs0, gs1, gs2, gs3)
    ssems = (ss0, ss1, ss2, ss3)
    _copy_node_rows(zeros_hbm, acc, s)
    pltpu.sync_copy(src_hbm.at[s], src_v)
    pltpu.sync_copy(dst_hbm.at[s], dst_v)
    plsc.subcore_barrier()

    def run(g_hbm):
      for b in range(3):
        pltpu.async_copy(g_hbm.at[src_v.at[b]], bufs[b], gsems[b])

      def body(k, _):
        for b in range(4):
          j = 4 * k + b
          b3 = (b + 3) % 4
          pltpu.make_async_copy(g_hbm.at[src_v.at[j]], bufs[b],
                                gsems[b]).wait()
          pltpu.async_copy(bufs[b], acc.at[dst_v.at[j]], ssems[b], add=True)
          @pl.when(j + 3 < E_ROWS_T)
          def _():
            @pl.when(j >= 1)
            def _():
              pltpu.make_async_copy(bufs[b3], acc.at[dst_v.at[j - 1]],
                                    ssems[b3]).wait()
            pltpu.async_copy(g_hbm.at[src_v.at[j + 3]], bufs[b3], gsems[b3])
        return ()
      lax.fori_loop(0, E_ROWS_T // 4, body, ())
      for b in range(4):
        j = E_ROWS_T - 4 + b
        pltpu.make_async_copy(bufs[b], acc.at[dst_v.at[j]], ssems[b]).wait()

    @pl.when(c == 0)
    def _():
      run(ga_hbm)
    @pl.when(c == 1)
    def _():
      run(gb_hbm)

    plsc.subcore_barrier()
    @pl.when(c == 0)
    def _():
      _copy_node_rows(acc, outa_hbm, s)
    @pl.when(c == 1)
    def _():
      _copy_node_rows(acc, outb_hbm, s)
  return segsum


_sc_segsum_64 = _make_sc_segsum(D_FEAT // 2)
_sc_segsum_32 = _make_sc_segsum(N_HID2 // 2)


@functools.partial(
    pl.kernel,
    out_type=jax.ShapeDtypeStruct((N_PAIRS, N_TYPES_PAD), jnp.float32),
    mesh=_mesh,
    scratch_types=[
        pltpu.VMEM((P_ROWS_W, PCHUNK), jnp.int32),
        pltpu.VMEM((P_ROWS_W, PCHUNK), jnp.int32),
        pltpu.VMEM((PCHUNK, N_TYPES_PAD), jnp.float32),
        pltpu.VMEM((PCHUNK, N_TYPES_PAD), jnp.float32),
        pltpu.VMEM((PCHUNK, N_TYPES_PAD), jnp.float32),
        pltpu.VMEM((PCHUNK, N_TYPES_PAD), jnp.float32),
        pltpu.VMEM((PCHUNK, N_TYPES_PAD), jnp.float32),
        pltpu.VMEM((PCHUNK, N_TYPES_PAD), jnp.float32),
        pltpu.SemaphoreType.DMA,
        pltpu.SemaphoreType.DMA,
        pltpu.SemaphoreType.DMA,
        pltpu.SemaphoreType.DMA,
        pltpu.SemaphoreType.DMA,
        pltpu.SemaphoreType.DMA,
    ],
    compiler_params=_sc_params,
)
def _sc_pairs(p1_hbm, p2_hbm, d1_hbm, d2_hbm, out_hbm,
              i1_v, i2_v, r1a, r2a, r1b, r2b, oba, obb,
              g1a, g2a, g1b, g2b, wsa, wsb):
  c, s, wid = _worker_id()
  pltpu.sync_copy(d1_hbm.at[wid], i1_v)
  pltpu.sync_copy(d2_hbm.at[wid], i2_v)
  base = wid * (P_ROWS_W * PCHUNK)

  def orows(j):
    return pl.ds(base + j * PCHUNK, PCHUNK)

  def gather(j, r1, r2, g1, g2):
    pltpu.async_copy(p1_hbm.at[i1_v.at[j]], r1, g1)
    pltpu.async_copy(p2_hbm.at[i2_v.at[j]], r2, g2)

  def slot(j, r1, r2, g1, g2, ob, ws, r1n, r2n, g1n, g2n):
    pltpu.make_async_copy(p1_hbm.at[i1_v.at[j]], r1, g1).wait()
    pltpu.make_async_copy(p2_hbm.at[i2_v.at[j]], r2, g2).wait()
    @pl.when(j + 1 < P_ROWS_W)
    def _():
      gather(j + 1, r1n, r2n, g1n, g2n)
    @pl.when(j >= 2)
    def _():
      pltpu.make_async_copy(ob, out_hbm.at[orows(j - 2)], ws).wait()

    def crow(r, _):
      for cc in range(N_TYPES_PAD // 16):
        a = r1[r, pl.ds(cc * 16, 16)]
        b = r2[r, pl.ds(cc * 16, 16)]
        z = a + b
        ob[r, pl.ds(cc * 16, 16)] = 1.0 / (1.0 + jnp.exp(-z))
      return ()
    lax.fori_loop(0, PCHUNK, crow, ())
    pltpu.async_copy(ob, out_hbm.at[orows(j)], ws)

  gather(0, r1a, r2a, g1a, g2a)

  def body(k, _):
    j0 = 2 * k
    slot(j0, r1a, r2a, g1a, g2a, oba, wsa, r1b, r2b, g1b, g2b)
    slot(j0 + 1, r1b, r2b, g1b, g2b, obb, wsb, r1a, r2a, g1a, g2a)
    return ()
  lax.fori_loop(0, P_ROWS_W // 2, body, ())
  slot(P_ROWS_W - 1, r1a, r2a, g1a, g2a, oba, wsa, r1b, r2b, g1b, g2b)
  pltpu.make_async_copy(obb, out_hbm.at[orows(P_ROWS_W - 2)], wsb).wait()
  pltpu.make_async_copy(oba, out_hbm.at[orows(P_ROWS_W - 1)], wsa).wait()


_BLK = 1000
_GRID = N_NODES // _BLK
_H = D_FEAT // 2
_Q = N_HID2 // 2


def _tc1_body(x_ref, w1_ref, p0_ref, p1_ref, ga_ref, gb_ref, dinv_ref):
  deg = 1.0 + p0_ref[:, 0:1] + p1_ref[:, 0:1]
  dinv = lax.rsqrt(deg)
  h = jnp.dot(x_ref[...], w1_ref[...], preferred_element_type=jnp.float32)
  g = h * dinv
  ga_ref[...] = g[:, :_H]
  gb_ref[...] = g[:, _H:]
  dinv_ref[...] = jnp.broadcast_to(dinv, (_BLK, D_FEAT))


def _tc2_body(sa_ref, sb_ref, ga_ref, gb_ref, dinv_ref, b1_ref, w2_ref,
              g2a_ref, g2b_ref):
  s = jnp.concatenate([sa_ref[...], sb_ref[...]], axis=1)
  g = jnp.concatenate([ga_ref[...], gb_ref[...]], axis=1)
  pre = dinv_ref[...] * (s + g) + b1_ref[...]
  h1 = jnp.maximum(pre, 0.0)
  g2 = jnp.dot(h1, w2_ref[...], preferred_element_type=jnp.float32)
  g2 = g2 * dinv_ref[:, :N_HID2]
  g2a_ref[...] = g2[:, :_Q]
  g2b_ref[...] = g2[:, _Q:]


def _tc3_body(sa_ref, sb_ref, g2a_ref, g2b_ref, dinv_ref, b2_ref,
              wf1_ref, wf2_ref, bf_ref, o1_ref, o2_ref):
  s = jnp.concatenate([sa_ref[...], sb_ref[...]], axis=1)
  g = jnp.concatenate([g2a_ref[...], g2b_ref[...]], axis=1)
  pre = dinv_ref[:, :N_HID2] * (s + g) + b2_ref[...]
  h2 = jnp.maximum(pre, 0.0)
  o1_ref[...] = jnp.dot(h2, wf1_ref[...],
                        preferred_element_type=jnp.float32) + bf_ref[...]
  o2_ref[...] = jnp.dot(h2, wf2_ref[...], preferred_element_type=jnp.float32)


def _row_spec(d):
  return pl.BlockSpec((_BLK, d), lambda i: (i, 0))


def _full_spec(r, d):
  return pl.BlockSpec((r, d), lambda i: (0, 0))


_tc1 = pl.pallas_call(
    _tc1_body,
    grid=(_GRID,),
    in_specs=[_row_spec(D_FEAT), _full_spec(D_FEAT, N_HID1),
              _row_spec(16), _row_spec(16)],
    out_specs=[_row_spec(_H), _row_spec(_H), _row_spec(D_FEAT)],
    out_shape=[jax.ShapeDtypeStruct((N_NODES, _H), jnp.float32),
               jax.ShapeDtypeStruct((N_NODES, _H), jnp.float32),
               jax.ShapeDtypeStruct((N_NODES, D_FEAT), jnp.float32)],
)

_tc2 = pl.pallas_call(
    _tc2_body,
    grid=(_GRID,),
    in_specs=[_row_spec(_H), _row_spec(_H), _row_spec(_H), _row_spec(_H),
              _row_spec(D_FEAT), _full_spec(1, N_HID1),
              _full_spec(N_HID1, N_HID2)],
    out_specs=[_row_spec(_Q), _row_spec(_Q)],
    out_shape=[jax.ShapeDtypeStruct((N_NODES, _Q), jnp.float32),
               jax.ShapeDtypeStruct((N_NODES, _Q), jnp.float32)],
)

_tc3 = pl.pallas_call(
    _tc3_body,
    grid=(_GRID,),
    in_specs=[_row_spec(_Q), _row_spec(_Q), _row_spec(_Q), _row_spec(_Q),
              _row_spec(D_FEAT),
              _full_spec(1, N_HID2),
              _full_spec(N_HID2, N_TYPES_PAD), _full_spec(N_HID2, N_TYPES_PAD),
              _full_spec(1, N_TYPES_PAD)],
    out_specs=[_row_spec(N_TYPES_PAD), _row_spec(N_TYPES_PAD)],
    out_shape=[jax.ShapeDtypeStruct((N_NODES, N_TYPES_PAD), jnp.float32),
               jax.ShapeDtypeStruct((N_NODES, N_TYPES_PAD), jnp.float32)],
)


@jax.jit
def kernel(x, W1, b1, W2, b2, Wf, bf, edge_index, drug1_idx, drug2_idx):
  dst_w = edge_index[1].reshape(NW, E_ROWS_W, ECHUNK)
  src_t = edge_index[0].reshape(NS, E_ROWS_T, ECHUNK)
  dst_t = edge_index[1].reshape(NS, E_ROWS_T, ECHUNK)
  d1_3d = drug1_idx.reshape(NW, P_ROWS_W, PCHUNK)
  d2_3d = drug2_idx.reshape(NW, P_ROWS_W, PCHUNK)

  ones16 = jnp.ones((ECHUNK, 16), jnp.float32)
  zeros16 = jnp.zeros((N_NODES, 16), jnp.float32)
  zeros_h = jnp.zeros((N_NODES, _H), jnp.float32)
  zeros_q = jnp.zeros((N_NODES, _Q), jnp.float32)

  wf1 = jnp.zeros((N_HID2, N_TYPES_PAD), jnp.float32).at[:, :N_TYPES].set(Wf[:N_HID2])
  wf2 = jnp.zeros((N_HID2, N_TYPES_PAD), jnp.float32).at[:, :N_TYPES].set(Wf[N_HID2:])
  bfp = jnp.zeros((1, N_TYPES_PAD), jnp.float32).at[0, :N_TYPES].set(bf)

  degp = _sc_degree(dst_w, ones16, zeros16)
  ga, gb, dinvb = _tc1(x, W1, degp[0], degp[1])
  s1a, s1b = _sc_segsum_64(ga, gb, src_t, dst_t, zeros_h)
  g2a, g2b = _tc2(s1a, s1b, ga, gb, dinvb, b1.reshape(1, N_HID1), W2)
  s2a, s2b = _sc_segsum_32(g2a, g2b, src_t, dst_t, zeros_q)
  p1, p2 = _tc3(s2a, s2b, g2a, g2b, dinvb, b2.reshape(1, N_HID2), wf1, wf2, bfp)
  out = _sc_pairs(p1, p2, d1_3d, d2_3d)
  return out[:, :N_TYPES]


# speedup vs baseline: 1.4690x; 1.0278x over previous
import functools

import jax
import jax.numpy as jnp
from jax import lax
from jax.experimental import pallas as pl
from jax.experimental.pallas import tpu as pltpu
from jax.experimental.pallas import tpu_sc as plsc

N_NODES = 10000
N_EDGES = 320000
D_FEAT = 128
N_HID1 = 128
N_HID2 = 64
N_TYPES = 86
N_TYPES_PAD = 96
N_PAIRS = 100000

NC = 2
NS = 16
NW = NC * NS
ECHUNK = 125
PCHUNK = 120
P_CHUNKS_W = 27
N_PAIRS_PAD = NW * P_CHUNKS_W * PCHUNK

E_ROWS_T = N_EDGES // ECHUNK // NS
E_ROWS_W = N_EDGES // ECHUNK // NW
P_ROWS_W = P_CHUNKS_W
NT_A = 624
NT_TAIL = N_NODES - NT_A * (NS - 1)

_mesh = plsc.VectorSubcoreMesh(core_axis_name="c", subcore_axis_name="s")
_sc_params = pltpu.CompilerParams(use_tc_tiling_on_sc=False)


def _worker_id():
  c = lax.axis_index("c")
  s = lax.axis_index("s")
  return c, s, c * NS + s


def _copy_node_rows(src, dst, s):
  @pl.when(s < NS - 1)
  def _():
    o = pl.multiple_of(s * NT_A, 8)
    pltpu.sync_copy(src.at[pl.ds(o, NT_A)], dst.at[pl.ds(o, NT_A)])
  @pl.when(s == NS - 1)
  def _():
    o = NT_A * (NS - 1)
    pltpu.sync_copy(src.at[pl.ds(o, NT_TAIL)], dst.at[pl.ds(o, NT_TAIL)])


@functools.partial(
    pl.kernel,
    out_type=jax.ShapeDtypeStruct((NC, N_NODES, 16), jnp.float32),
    mesh=_mesh,
    scratch_types=[
        pltpu.VMEM((E_ROWS_W, ECHUNK), jnp.int32),
        pltpu.VMEM((ECHUNK, 16), jnp.float32),
        pltpu.VMEM_SHARED((N_NODES, 16), jnp.float32),
    ],
    compiler_params=_sc_params,
)
def _sc_degree(dst_hbm, ones_hbm, zeros_hbm, out_hbm, idx_v, ones_v, acc):
  c, s, wid = _worker_id()
  _copy_node_rows(zeros_hbm, acc, s)
  pltpu.sync_copy(ones_hbm, ones_v)
  pltpu.sync_copy(dst_hbm.at[wid], idx_v)
  plsc.subcore_barrier()

  def body(j, _):
    pltpu.sync_copy(ones_v, acc.at[idx_v.at[j]], add=True)
    return ()
  lax.fori_loop(0, E_ROWS_W, body, ())

  plsc.subcore_barrier()
  _copy_node_rows(acc, out_hbm.at[c], s)


def _make_sc_segsum(d):
  @functools.partial(
      pl.kernel,
      out_type=(jax.ShapeDtypeStruct((N_NODES, d), jnp.float32),
                jax.ShapeDtypeStruct((N_NODES, d), jnp.float32)),
      mesh=_mesh,
      scratch_types=[
          pltpu.VMEM((E_ROWS_T, ECHUNK), jnp.int32),
          pltpu.VMEM((E_ROWS_T, ECHUNK), jnp.int32),
          pltpu.VMEM((ECHUNK, d), jnp.float32),
          pltpu.VMEM((ECHUNK, d), jnp.float32),
          pltpu.VMEM((ECHUNK, d), jnp.float32),
          pltpu.VMEM((ECHUNK, d), jnp.float32),
          pltpu.VMEM_SHARED((N_NODES, d), jnp.float32),
          pltpu.SemaphoreType.DMA,
          pltpu.SemaphoreType.DMA,
          pltpu.SemaphoreType.DMA,
          pltpu.SemaphoreType.DMA,
          pltpu.SemaphoreType.DMA,
          pltpu.SemaphoreType.DMA,
          pltpu.SemaphoreType.DMA,
          pltpu.SemaphoreType.DMA,
      ],
      compiler_params=_sc_params,
  )
  def segsum(ga_hbm, gb_hbm, src_hbm, dst_hbm, zeros_hbm, outa_hbm, outb_hbm,
             src_v, dst_v, buf0, buf1, buf2, buf3, acc,
             gs0, gs1, gs2, gs3, ss0, ss1, ss2, ss3):
    c, s, wid = _worker_id()
    bufs = (buf0, buf1, buf2, buf3)
    gsems = (gs0, gs1, gs2, gs3)
    ssems = (ss0, ss1, ss2, ss3)
    _copy_node_rows(zeros_hbm, acc, s)
    pltpu.sync_copy(src_hbm.at[s], src_v)
    pltpu.sync_copy(dst_hbm.at[s], dst_v)
    plsc.subcore_barrier()

    def run(g_hbm):
      for b in range(3):
        pltpu.async_copy(g_hbm.at[src_v.at[b]], bufs[b], gsems[b])

      def body(k, _):
        for b in range(4):
          j = 4 * k + b
          b3 = (b + 3) % 4
          pltpu.make_async_copy(g_hbm.at[src_v.at[j]], bufs[b],
                                gsems[b]).wait()
          pltpu.async_copy(bufs[b], acc.at[dst_v.at[j]], ssems[b], add=True)
          @pl.when(j + 3 < E_ROWS_T)
          def _():
            @pl.when(j >= 1)
            def _():
              pltpu.make_async_copy(bufs[b3], acc.at[dst_v.at[j - 1]],
                                    ssems[b3]).wait()
            pltpu.async_copy(g_hbm.at[src_v.at[j + 3]], bufs[b3], gsems[b3])
        return ()
      lax.fori_loop(0, E_ROWS_T // 4, body, ())
      for b in range(4):
        j = E_ROWS_T - 4 + b
        pltpu.make_async_copy(bufs[b], acc.at[dst_v.at[j]], ssems[b]).wait()

    @pl.when(c == 0)
    def _():
      run(ga_hbm)
    @pl.when(c == 1)
    def _():
      run(gb_hbm)

    plsc.subcore_barrier()
    @pl.when(c == 0)
    def _():
      _copy_node_rows(acc, outa_hbm, s)
    @pl.when(c == 1)
    def _():
      _copy_node_rows(acc, outb_hbm, s)
  return segsum


_sc_segsum_64 = _make_sc_segsum(D_FEAT // 2)
_sc_segsum_32 = _make_sc_segsum(N_HID2 // 2)


@functools.partial(
    pl.kernel,
    out_type=jax.ShapeDtypeStruct((N_PAIRS_PAD, 128), jnp.float32),
    mesh=_mesh,
    scratch_types=[
        pltpu.VMEM((P_ROWS_W, PCHUNK), jnp.int32),
        pltpu.VMEM((P_ROWS_W, PCHUNK), jnp.int32),
        pltpu.VMEM((PCHUNK, 128), jnp.float32),
        pltpu.VMEM((PCHUNK, 128), jnp.float32),
        pltpu.VMEM((PCHUNK, 128), jnp.float32),
        pltpu.VMEM((PCHUNK, 128), jnp.float32),
        pltpu.VMEM((PCHUNK, 128), jnp.float32),
        pltpu.VMEM((PCHUNK, 128), jnp.float32),
        pltpu.SemaphoreType.DMA,
        pltpu.SemaphoreType.DMA,
        pltpu.SemaphoreType.DMA,
        pltpu.SemaphoreType.DMA,
        pltpu.SemaphoreType.DMA,
        pltpu.SemaphoreType.DMA,
    ],
)
def _sc_pairs(p1_hbm, p2_hbm, d1_hbm, d2_hbm, out_hbm,
              i1_v, i2_v, r1a, r2a, r1b, r2b, oba, obb,
              g1a, g2a, g1b, g2b, wsa, wsb):
  c, s, wid = _worker_id()
  pltpu.sync_copy(d1_hbm.at[wid], i1_v)
  pltpu.sync_copy(d2_hbm.at[wid], i2_v)
  base = wid * (P_ROWS_W * PCHUNK)

  def orows(j):
    return pl.ds(pl.multiple_of(base + j * PCHUNK, 8), PCHUNK)

  def gather(j, r1, r2, g1, g2):
    pltpu.async_copy(p1_hbm.at[i1_v.at[j]], r1, g1)
    pltpu.async_copy(p2_hbm.at[i2_v.at[j]], r2, g2)

  def slot(j, r1, r2, g1, g2, ob, ws, r1n, r2n, g1n, g2n):
    pltpu.make_async_copy(p1_hbm.at[i1_v.at[j]], r1, g1).wait()
    pltpu.make_async_copy(p2_hbm.at[i2_v.at[j]], r2, g2).wait()
    @pl.when(j + 1 < P_ROWS_W)
    def _():
      gather(j + 1, r1n, r2n, g1n, g2n)
    @pl.when(j >= 2)
    def _():
      pltpu.make_async_copy(ob, out_hbm.at[orows(j - 2)], ws).wait()

    def crow(r, _):
      for cc in range(N_TYPES_PAD // 16):
        a = r1[r, pl.ds(cc * 16, 16)]
        b = r2[r, pl.ds(cc * 16, 16)]
        z = a + b
        ob[r, pl.ds(cc * 16, 16)] = 1.0 / (1.0 + jnp.exp(-z))
      return ()
    lax.fori_loop(0, PCHUNK, crow, ())
    pltpu.async_copy(ob, out_hbm.at[orows(j)], ws)

  gather(0, r1a, r2a, g1a, g2a)

  def body(k, _):
    j0 = 2 * k
    slot(j0, r1a, r2a, g1a, g2a, oba, wsa, r1b, r2b, g1b, g2b)
    slot(j0 + 1, r1b, r2b, g1b, g2b, obb, wsb, r1a, r2a, g1a, g2a)
    return ()
  lax.fori_loop(0, P_ROWS_W // 2, body, ())
  slot(P_ROWS_W - 1, r1a, r2a, g1a, g2a, oba, wsa, r1b, r2b, g1b, g2b)
  pltpu.make_async_copy(obb, out_hbm.at[orows(P_ROWS_W - 2)], wsb).wait()
  pltpu.make_async_copy(oba, out_hbm.at[orows(P_ROWS_W - 1)], wsa).wait()


_BLK = 1000
_GRID = N_NODES // _BLK
_H = D_FEAT // 2
_Q = N_HID2 // 2


def _tc1_body(x_ref, w1_ref, p0_ref, p1_ref, ga_ref, gb_ref, dinv_ref):
  deg = 1.0 + p0_ref[:, 0:1] + p1_ref[:, 0:1]
  dinv = lax.rsqrt(deg)
  h = jnp.dot(x_ref[...], w1_ref[...], preferred_element_type=jnp.float32)
  g = h * dinv
  ga_ref[...] = g[:, :_H]
  gb_ref[...] = g[:, _H:]
  dinv_ref[...] = jnp.broadcast_to(dinv, (_BLK, D_FEAT))


def _tc2_body(sa_ref, sb_ref, ga_ref, gb_ref, dinv_ref, b1_ref, w2_ref,
              g2a_ref, g2b_ref):
  s = jnp.concatenate([sa_ref[...], sb_ref[...]], axis=1)
  g = jnp.concatenate([ga_ref[...], gb_ref[...]], axis=1)
  pre = dinv_ref[...] * (s + g) + b1_ref[...]
  h1 = jnp.maximum(pre, 0.0)
  g2 = jnp.dot(h1, w2_ref[...], preferred_element_type=jnp.float32)
  g2 = g2 * dinv_ref[:, :N_HID2]
  g2a_ref[...] = g2[:, :_Q]
  g2b_ref[...] = g2[:, _Q:]


def _tc3_body(sa_ref, sb_ref, g2a_ref, g2b_ref, dinv_ref, b2_ref,
              wf1_ref, wf2_ref, bf_ref, o1_ref, o2_ref):
  s = jnp.concatenate([sa_ref[...], sb_ref[...]], axis=1)
  g = jnp.concatenate([g2a_ref[...], g2b_ref[...]], axis=1)
  pre = dinv_ref[:, :N_HID2] * (s + g) + b2_ref[...]
  h2 = jnp.maximum(pre, 0.0)
  o1_ref[...] = jnp.dot(h2, wf1_ref[...],
                        preferred_element_type=jnp.float32) + bf_ref[...]
  o2_ref[...] = jnp.dot(h2, wf2_ref[...], preferred_element_type=jnp.float32)


def _row_spec(d):
  return pl.BlockSpec((_BLK, d), lambda i: (i, 0))


def _full_spec(r, d):
  return pl.BlockSpec((r, d), lambda i: (0, 0))


_tc1 = pl.pallas_call(
    _tc1_body,
    grid=(_GRID,),
    in_specs=[_row_spec(D_FEAT), _full_spec(D_FEAT, N_HID1),
              _row_spec(16), _row_spec(16)],
    out_specs=[_row_spec(_H), _row_spec(_H), _row_spec(D_FEAT)],
    out_shape=[jax.ShapeDtypeStruct((N_NODES, _H), jnp.float32),
               jax.ShapeDtypeStruct((N_NODES, _H), jnp.float32),
               jax.ShapeDtypeStruct((N_NODES, D_FEAT), jnp.float32)],
)

_tc2 = pl.pallas_call(
    _tc2_body,
    grid=(_GRID,),
    in_specs=[_row_spec(_H), _row_spec(_H), _row_spec(_H), _row_spec(_H),
              _row_spec(D_FEAT), _full_spec(1, N_HID1),
              _full_spec(N_HID1, N_HID2)],
    out_specs=[_row_spec(_Q), _row_spec(_Q)],
    out_shape=[jax.ShapeDtypeStruct((N_NODES, _Q), jnp.float32),
               jax.ShapeDtypeStruct((N_NODES, _Q), jnp.float32)],
)

_tc3 = pl.pallas_call(
    _tc3_body,
    grid=(_GRID,),
    in_specs=[_row_spec(_Q), _row_spec(_Q), _row_spec(_Q), _row_spec(_Q),
              _row_spec(D_FEAT),
              _full_spec(1, N_HID2),
              _full_spec(N_HID2, 128), _full_spec(N_HID2, 128),
              _full_spec(1, 128)],
    out_specs=[_row_spec(128), _row_spec(128)],
    out_shape=[jax.ShapeDtypeStruct((N_NODES, 128), jnp.float32),
               jax.ShapeDtypeStruct((N_NODES, 128), jnp.float32)],
)


@jax.jit
def kernel(x, W1, b1, W2, b2, Wf, bf, edge_index, drug1_idx, drug2_idx):
  dst_w = edge_index[1].reshape(NW, E_ROWS_W, ECHUNK)
  src_t = edge_index[0].reshape(NS, E_ROWS_T, ECHUNK)
  dst_t = edge_index[1].reshape(NS, E_ROWS_T, ECHUNK)
  npad = N_PAIRS_PAD - N_PAIRS
  d1_3d = jnp.pad(drug1_idx, (0, npad), mode="wrap").reshape(NW, P_ROWS_W, PCHUNK)
  d2_3d = jnp.pad(drug2_idx, (0, npad), mode="wrap").reshape(NW, P_ROWS_W, PCHUNK)

  ones16 = jnp.ones((ECHUNK, 16), jnp.float32)
  zeros16 = jnp.zeros((N_NODES, 16), jnp.float32)
  zeros_h = jnp.zeros((N_NODES, _H), jnp.float32)
  zeros_q = jnp.zeros((N_NODES, _Q), jnp.float32)

  wf1 = jnp.zeros((N_HID2, 128), jnp.float32).at[:, :N_TYPES].set(Wf[:N_HID2])
  wf2 = jnp.zeros((N_HID2, 128), jnp.float32).at[:, :N_TYPES].set(Wf[N_HID2:])
  bfp = jnp.zeros((1, 128), jnp.float32).at[0, :N_TYPES].set(bf)

  degp = _sc_degree(dst_w, ones16, zeros16)
  ga, gb, dinvb = _tc1(x, W1, degp[0], degp[1])
  s1a, s1b = _sc_segsum_64(ga, gb, src_t, dst_t, zeros_h)
  g2a, g2b = _tc2(s1a, s1b, ga, gb, dinvb, b1.reshape(1, N_HID1), W2)
  s2a, s2b = _sc_segsum_32(g2a, g2b, src_t, dst_t, zeros_q)
  p1, p2 = _tc3(s2a, s2b, g2a, g2b, dinvb, b2.reshape(1, N_HID2), wf1, wf2, bfp)
  out = _sc_pairs(p1, p2, d1_3d, d2_3d)
  return out[:N_PAIRS, :N_TYPES]
